# Initial kernel scaffold; baseline (speedup 1.0000x reference)
import jax
import jax.numpy as jnp
from jax.experimental import pallas as pl


def kernel(x, edge_index, enc_W1, enc_b1, enc_W2, enc_b2, enc_W3, enc_b3, conv_W, conv_att_src, conv_att_dst, conv_b, dec_W1, dec_b1, dec_W2, dec_b2, dec_W3, dec_b3):
    raise NotImplementedError("write your pallas kernel here")


# speedup vs baseline: 1.2297x; 1.2297x over previous
import functools
import jax
import jax.numpy as jnp
from jax.experimental import pallas as pl
from jax.experimental.pallas import tpu as pltpu

_BN = 2000


def _enc_body(x_ref, W1, b1, W2, b2, W3, b3, Wc, was, wad,
              y_ref, as_ref, ad_ref):
    h = jnp.maximum(jnp.dot(x_ref[...], W1[...],
                            preferred_element_type=jnp.float32) + b1[...], 0.0)
    h = jnp.maximum(jnp.dot(h, W2[...],
                            preferred_element_type=jnp.float32) + b2[...], 0.0)
    h = jnp.dot(h, W3[...], preferred_element_type=jnp.float32) + b3[...]
    y = jnp.dot(h, Wc[...], preferred_element_type=jnp.float32)
    y_ref[...] = y
    as_ref[...] = (y @ was[...])[:, None]
    ad_ref[...] = (y @ wad[...])[:, None]


def _mid_body(agg_ref, bprev, W, was, wad, y_ref, as_ref, ad_ref):
    g = jnp.maximum(agg_ref[...] + bprev[...], 0.0)
    y = jnp.dot(g, W[...], preferred_element_type=jnp.float32)
    y_ref[...] = y
    as_ref[...] = (y @ was[...])[:, None]
    ad_ref[...] = (y @ wad[...])[:, None]


def _dec_body(agg_ref, bprev, W1, b1, W2, b2, W3, b3, out_ref):
    g = jnp.maximum(agg_ref[...] + bprev[...], 0.0)
    h = jnp.maximum(jnp.dot(g, W1[...],
                            preferred_element_type=jnp.float32) + b1[...], 0.0)
    h = jnp.maximum(jnp.dot(h, W2[...],
                            preferred_element_type=jnp.float32) + b2[...], 0.0)
    h = jnp.dot(h, W3[...], preferred_element_type=jnp.float32) + b3[...]
    out_ref[...] = jax.nn.sigmoid(h)


def _full2d(shape):
    return pl.BlockSpec(shape, lambda i: (0, 0))


def _full1d(n):
    return pl.BlockSpec((n,), lambda i: (0,))


def _row_spec(bn, d):
    return pl.BlockSpec((bn, d), lambda i: (i, 0))


def _vec_spec(bn):
    return pl.BlockSpec((bn, 1), lambda i: (i, 0))


def _enc_call(x, W1, b1, W2, b2, W3, b3, Wc, was, wad):
    n, d = x.shape
    h = W1.shape[1]
    grid = (n // _BN,)
    return pl.pallas_call(
        _enc_body,
        grid=grid,
        in_specs=[_row_spec(_BN, d), _full2d((d, h)), _full1d(h),
                  _full2d((h, h)), _full1d(h), _full2d((h, h)), _full1d(h),
                  _full2d((h, h)), _full1d(h), _full1d(h)],
        out_specs=[_row_spec(_BN, h), _vec_spec(_BN), _vec_spec(_BN)],
        out_shape=[jax.ShapeDtypeStruct((n, h), jnp.float32),
                   jax.ShapeDtypeStruct((n, 1), jnp.float32),
                   jax.ShapeDtypeStruct((n, 1), jnp.float32)],
    )(x, W1, b1, W2, b2, W3, b3, Wc, was, wad)


def _mid_call(agg, bprev, W, was, wad):
    n, h = agg.shape
    grid = (n // _BN,)
    return pl.pallas_call(
        _mid_body,
        grid=grid,
        in_specs=[_row_spec(_BN, h), _full1d(h), _full2d((h, h)),
                  _full1d(h), _full1d(h)],
        out_specs=[_row_spec(_BN, h), _vec_spec(_BN), _vec_spec(_BN)],
        out_shape=[jax.ShapeDtypeStruct((n, h), jnp.float32),
                   jax.ShapeDtypeStruct((n, 1), jnp.float32),
                   jax.ShapeDtypeStruct((n, 1), jnp.float32)],
    )(agg, bprev, W, was, wad)


def _dec_call(agg, bprev, W1, b1, W2, b2, W3, b3):
    n, h = agg.shape
    d = W3.shape[1]
    grid = (n // _BN,)
    return pl.pallas_call(
        _dec_body,
        grid=grid,
        in_specs=[_row_spec(_BN, h), _full1d(h), _full2d((h, h)), _full1d(h),
                  _full2d((h, h)), _full1d(h), _full2d((h, d)), _full1d(d)],
        out_specs=[_row_spec(_BN, d)],
        out_shape=[jax.ShapeDtypeStruct((n, d), jnp.float32)],
    )(agg, bprev, W1, b1, W2, b2, W3, b3)[0]


def _edge_phase(y, a_s, a_d, src, dst, n):
    m = jax.nn.leaky_relu(jnp.max(a_s) + jnp.max(a_d), 0.2)
    e = a_s[src] + a_d[dst]
    e = jax.nn.leaky_relu(e, 0.2)
    ex = jnp.exp(e - m)
    denom = jax.ops.segment_sum(ex, dst, num_segments=n)
    coef = ex / (denom[dst] + 1e-16)
    return jax.ops.segment_sum(coef[:, None] * y[src], dst, num_segments=n)


def kernel(x, edge_index, enc_W1, enc_b1, enc_W2, enc_b2, enc_W3, enc_b3,
           conv_W, conv_att_src, conv_att_dst, conv_b,
           dec_W1, dec_b1, dec_W2, dec_b2, dec_W3, dec_b3):
    n = x.shape[0]
    L = conv_W.shape[0]
    loop = jnp.arange(n, dtype=edge_index.dtype)
    src = jnp.concatenate([edge_index[0], loop])
    dst = jnp.concatenate([edge_index[1], loop])

    y, a_s, a_d = _enc_call(x, enc_W1, enc_b1, enc_W2, enc_b2, enc_W3, enc_b3,
                            conv_W[0], conv_att_src[0], conv_att_dst[0])
    a_s, a_d = a_s[:, 0], a_d[:, 0]
    for i in range(L):
        agg = _edge_phase(y, a_s, a_d, src, dst, n)
        if i < L - 1:
            y, a_s, a_d = _mid_call(agg, conv_b[i], conv_W[i + 1],
                                    conv_att_src[i + 1], conv_att_dst[i + 1])
            a_s, a_d = a_s[:, 0], a_d[:, 0]
    return _dec_call(agg, conv_b[L - 1], dec_W1, dec_b1, dec_W2, dec_b2,
                     dec_W3, dec_b3)


# speedup vs baseline: 9.2542x; 7.5258x over previous
import functools
import jax
import jax.numpy as jnp
from jax import lax
from jax.experimental import pallas as pl
from jax.experimental.pallas import tpu as pltpu
from jax.experimental.pallas import tpu_sc as plsc

_BN = 2000
_NPAD = 10240
_TILES = 16
_CW = 128
_Q = 64


def _emit_y(y, was, wad, y_refs, as_ref, ad_ref, ms_ref, md_ref):
    for q in range(4):
        y_refs[q][...] = y[:, q * _Q:(q + 1) * _Q]
    i = pl.program_id(0)
    asb = y @ was[...]
    adb = y @ wad[...]
    as_ref[...] = asb[:, None]
    ad_ref[...] = adb[:, None]
    prev_s = jnp.where(i == 0, jnp.full((1, 1), -jnp.inf, jnp.float32),
                       ms_ref[...])
    prev_d = jnp.where(i == 0, jnp.full((1, 1), -jnp.inf, jnp.float32),
                       md_ref[...])
    ms_ref[...] = jnp.maximum(prev_s, jnp.max(asb))
    md_ref[...] = jnp.maximum(prev_d, jnp.max(adb))


def _enc_body(x_ref, W1, b1, W2, b2, W3, b3, Wc, was, wad,
              y0, y1, y2, y3, as_ref, ad_ref, ms_ref, md_ref):
    h = jnp.maximum(jnp.dot(x_ref[...], W1[...],
                            preferred_element_type=jnp.float32) + b1[...], 0.0)
    h = jnp.maximum(jnp.dot(h, W2[...],
                            preferred_element_type=jnp.float32) + b2[...], 0.0)
    h = jnp.dot(h, W3[...], preferred_element_type=jnp.float32) + b3[...]
    y = jnp.dot(h, Wc[...], preferred_element_type=jnp.float32)
    _emit_y(y, was, wad, (y0, y1, y2, y3), as_ref, ad_ref, ms_ref, md_ref)


def _mid_body(a0, a1, a2, a3, bprev, W, was, wad,
              y0, y1, y2, y3, as_ref, ad_ref, ms_ref, md_ref):
    g = jnp.concatenate([a0[...], a1[...], a2[...], a3[...]], axis=1)
    g = jnp.maximum(g + bprev[...], 0.0)
    y = jnp.dot(g, W[...], preferred_element_type=jnp.float32)
    _emit_y(y, was, wad, (y0, y1, y2, y3), as_ref, ad_ref, ms_ref, md_ref)


def _dec_body(a0, a1, a2, a3, bprev, W1, b1, W2, b2, W3, b3, out_ref):
    g = jnp.concatenate([a0[...], a1[...], a2[...], a3[...]], axis=1)
    g = jnp.maximum(g + bprev[...], 0.0)
    h = jnp.maximum(jnp.dot(g, W1[...],
                            preferred_element_type=jnp.float32) + b1[...], 0.0)
    h = jnp.maximum(jnp.dot(h, W2[...],
                            preferred_element_type=jnp.float32) + b2[...], 0.0)
    h = jnp.dot(h, W3[...], preferred_element_type=jnp.float32) + b3[...]
    out_ref[...] = jax.nn.sigmoid(h)


def _full2d(shape):
    return pl.BlockSpec(shape, lambda i: (0, 0))


def _full1d(n):
    return pl.BlockSpec((n,), lambda i: (0,))


def _row_spec(bn, d):
    return pl.BlockSpec((bn, d), lambda i: (i, 0))


def _vec_spec(bn):
    return pl.BlockSpec((bn, 1), lambda i: (i, 0))


def _scalar_spec():
    return pl.BlockSpec((1, 1), lambda i: (0, 0))


def _y_out_specs(n):
    qs = [_row_spec(_BN, _Q)] * 4
    qt = [jax.ShapeDtypeStruct((n, _Q), jnp.float32)] * 4
    return (qs + [_vec_spec(_BN), _vec_spec(_BN),
                  _scalar_spec(), _scalar_spec()],
            qt + [jax.ShapeDtypeStruct((n, 1), jnp.float32),
                  jax.ShapeDtypeStruct((n, 1), jnp.float32),
                  jax.ShapeDtypeStruct((1, 1), jnp.float32),
                  jax.ShapeDtypeStruct((1, 1), jnp.float32)])


def _enc_call(x, W1, b1, W2, b2, W3, b3, Wc, was, wad):
    n, d = x.shape
    h = W1.shape[1]
    out_specs, out_shape = _y_out_specs(n)
    return pl.pallas_call(
        _enc_body,
        grid=(n // _BN,),
        in_specs=[_row_spec(_BN, d), _full2d((d, h)), _full1d(h),
                  _full2d((h, h)), _full1d(h), _full2d((h, h)), _full1d(h),
                  _full2d((h, h)), _full1d(h), _full1d(h)],
        out_specs=out_specs,
        out_shape=out_shape,
    )(x, W1, b1, W2, b2, W3, b3, Wc, was, wad)


def _mid_call(aq, bprev, W, was, wad):
    n = aq[0].shape[0]
    h = W.shape[0]
    out_specs, out_shape = _y_out_specs(n)
    return pl.pallas_call(
        _mid_body,
        grid=(n // _BN,),
        in_specs=[_row_spec(_BN, _Q)] * 4 + [_full1d(h), _full2d((h, h)),
                                             _full1d(h), _full1d(h)],
        out_specs=out_specs,
        out_shape=out_shape,
    )(*aq, bprev, W, was, wad)


def _dec_call(aq, bprev, W1, b1, W2, b2, W3, b3):
    n = aq[0].shape[0]
    h = W1.shape[0]
    d = W3.shape[1]
    return pl.pallas_call(
        _dec_body,
        grid=(n // _BN,),
        in_specs=[_row_spec(_BN, _Q)] * 4 + [
            _full1d(h), _full2d((h, h)), _full1d(h),
            _full2d((h, h)), _full1d(h), _full2d((h, d)), _full1d(d)],
        out_specs=[_row_spec(_BN, d)],
        out_shape=[jax.ShapeDtypeStruct((n, d), jnp.float32)],
    )(*aq, bprev, W1, b1, W2, b2, W3, b3)[0]


def _sc_edge(yq, asp, adp, mvec, srcb, dstb):
    ct = srcb.shape[1]
    rpt = _NPAD // _TILES
    mesh = plsc.VectorSubcoreMesh(core_axis_name="c", subcore_axis_name="s")

    @functools.partial(
        pl.kernel, mesh=mesh,
        out_type=jax.ShapeDtypeStruct((4, _NPAD, _Q), jnp.float32),
        compiler_params=pltpu.CompilerParams(needs_layout_passes=False,
                                             use_tc_tiling_on_sc=False),
        scratch_types=[
            pltpu.VMEM((_NPAD,), jnp.float32),
            pltpu.VMEM((_NPAD,), jnp.float32),
            pltpu.VMEM((16,), jnp.float32),
            pltpu.VMEM((ct, _CW), jnp.int32),
            pltpu.VMEM((ct, _CW), jnp.int32),
            pltpu.VMEM((ct, _CW), jnp.float32),
            pltpu.VMEM((_CW, _Q), jnp.float32),
            pltpu.VMEM((_NPAD,), jnp.float32),
            pltpu.VMEM((rpt,), jnp.float32),
            pltpu.VMEM_SHARED((_NPAD, _Q), jnp.float32),
            pltpu.VMEM_SHARED((_NPAD,), jnp.float32),
            pltpu.SemaphoreType.DMA,
        ])
    def k(y0_h, y1_h, y2_h, y3_h, as_h, ad_h, m_h, srcb_h, dstb_h, out_h,
          asrc_v, adst_v, m_v, src_v, dst_v, ex_v, rows_v, den_v,
          zden_v, out_sh, den_sh, sem):
        c = lax.axis_index("c")
        s = lax.axis_index("s")
        pltpu.sync_copy(as_h, asrc_v)
        pltpu.sync_copy(ad_h, adst_v)
        pltpu.sync_copy(m_h, m_v)
        pltpu.sync_copy(srcb_h.at[s], src_v)
        pltpu.sync_copy(dstb_h.at[s], dst_v)

        zero16 = jnp.zeros((16,), jnp.float32)
        base = s * rpt

        def zrows(j, carry):
            for kk in range(_Q // 16):
                rows_v[j, pl.ds(kk * 16, 16)] = zero16
            return carry

        def zero_my_slice():
            lax.fori_loop(0, _CW, zrows, 0)
            for t in range(rpt // _CW):
                pltpu.sync_copy(rows_v, out_sh.at[pl.ds(base + t * _CW, _CW)])

        zero_my_slice()
        for t in range(rpt // 16):
            zden_v[pl.ds(t * 16, 16)] = zero16
        pltpu.sync_copy(zden_v, den_sh.at[pl.ds(base, rpt)])
        plsc.subcore_barrier()

        mraw = m_v[...]
        mm = jnp.maximum(mraw, 0.2 * mraw)

        def ph1(g, carry):
            for kk in range(8):
                si = src_v[g, pl.ds(kk * 16, 16)]
                di = dst_v[g, pl.ds(kk * 16, 16)]
                e = (plsc.load_gather(asrc_v, [si]) +
                     plsc.load_gather(adst_v, [di]))
                e = jnp.maximum(e, 0.2 * e)
                ex_v[g, pl.ds(kk * 16, 16)] = jnp.exp(e - mm)
            pltpu.sync_copy(ex_v.at[g], den_sh.at[dst_v.at[g]], add=True)
            return carry
        lax.fori_loop(0, ct, ph1, 0)
        plsc.subcore_barrier()
        pltpu.sync_copy(den_sh, den_v)

        def agg_pass(y_ref, first):
            def body(g, carry):
                if first:
                    for kk in range(8):
                        di = dst_v[g, pl.ds(kk * 16, 16)]
                        dg = plsc.load_gather(den_v, [di])
                        ex_v[g, pl.ds(kk * 16, 16)] = (
                            ex_v[g, pl.ds(kk * 16, 16)] / (dg + 1e-16))
                pltpu.async_copy(y_ref.at[src_v.at[g]], rows_v, sem).wait()

                def scale(j, cc):
                    gs = jnp.broadcast_to(g, (16,)).astype(jnp.int32)
                    js = jnp.broadcast_to(j, (16,)).astype(jnp.int32)
                    cf = plsc.load_gather(ex_v, [gs, js])
                    for kk in range(_Q // 16):
                        rows_v[j, pl.ds(kk * 16, 16)] = (
                            rows_v[j, pl.ds(kk * 16, 16)] * cf)
                    return cc
                lax.fori_loop(0, _CW, scale, 0)
                pltpu.sync_copy(rows_v, out_sh.at[dst_v.at[g]], add=True)
                return carry
            lax.fori_loop(0, ct, body, 0)

        def export(qi):
            pltpu.sync_copy(out_sh.at[pl.ds(base, rpt)],
                            out_h.at[qi, pl.ds(base, rpt)])

        def two_quarters(y_lo, y_hi, qi0):
            agg_pass(y_lo, True)
            plsc.subcore_barrier()
            export(qi0)
            zero_my_slice()
            plsc.subcore_barrier()
            agg_pass(y_hi, False)
            plsc.subcore_barrier()
            export(qi0 + 1)

        @pl.when(c == 0)
        def _():
            two_quarters(y0_h, y1_h, 0)

        @pl.when(c == 1)
        def _():
            two_quarters(y2_h, y3_h, 2)

    return k(*yq, asp, adp, mvec, srcb, dstb)


def kernel(x, edge_index, enc_W1, enc_b1, enc_W2, enc_b2, enc_W3, enc_b3,
           conv_W, conv_att_src, conv_att_dst, conv_b,
           dec_W1, dec_b1, dec_W2, dec_b2, dec_W3, dec_b3):
    n = x.shape[0]
    L = conv_W.shape[0]
    loop = jnp.arange(n, dtype=edge_index.dtype)
    src = jnp.concatenate([edge_index[0], loop]).astype(jnp.int32)
    dst = jnp.concatenate([edge_index[1], loop]).astype(jnp.int32)
    ne = src.shape[0]
    ct = -(-ne // (_TILES * _CW))
    epad = _TILES * ct * _CW
    srcb = jnp.pad(src, (0, epad - ne)).reshape(_TILES, ct, _CW)
    dstb = jnp.pad(dst, (0, epad - ne),
                   constant_values=n).reshape(_TILES, ct, _CW)

    *yq, a_s, a_d, ms, md = _enc_call(
        x, enc_W1, enc_b1, enc_W2, enc_b2, enc_W3, enc_b3,
        conv_W[0], conv_att_src[0], conv_att_dst[0])
    for i in range(L):
        asp = jnp.pad(a_s[:, 0], (0, _NPAD - n))
        adp = jnp.pad(a_d[:, 0], (0, _NPAD - n))
        mvec = jnp.full((16,), ms[0, 0] + md[0, 0], jnp.float32)
        out_q = _sc_edge(yq, asp, adp, mvec, srcb, dstb)
        aq = [out_q[q, :n] for q in range(4)]
        if i < L - 1:
            *yq, a_s, a_d, ms, md = _mid_call(
                aq, conv_b[i], conv_W[i + 1],
                conv_att_src[i + 1], conv_att_dst[i + 1])
    return _dec_call(aq, conv_b[L - 1], dec_W1, dec_b1, dec_W2, dec_b2,
                     dec_W3, dec_b3)


# speedup vs baseline: 10.4178x; 1.1257x over previous
import functools
import jax
import jax.numpy as jnp
from jax import lax
from jax.experimental import pallas as pl
from jax.experimental.pallas import tpu as pltpu
from jax.experimental.pallas import tpu_sc as plsc

_BN = 2000
_NPAD = 10240
_TILES = 16
_CW = 128
_Q = 64


def _emit_y(y, was, wad, y_refs, as_ref, ad_ref, ms_ref, md_ref):
    for q in range(4):
        y_refs[q][...] = y[:, q * _Q:(q + 1) * _Q]
    i = pl.program_id(0)
    asb = y @ was[...]
    adb = y @ wad[...]
    as_ref[...] = asb[:, None]
    ad_ref[...] = adb[:, None]
    prev_s = jnp.where(i == 0, jnp.full((1, 1), -jnp.inf, jnp.float32),
                       ms_ref[...])
    prev_d = jnp.where(i == 0, jnp.full((1, 1), -jnp.inf, jnp.float32),
                       md_ref[...])
    ms_ref[...] = jnp.maximum(prev_s, jnp.max(asb))
    md_ref[...] = jnp.maximum(prev_d, jnp.max(adb))


def _enc_body(x_ref, W1, b1, W2, b2, W3, b3, Wc, was, wad,
              y0, y1, y2, y3, as_ref, ad_ref, ms_ref, md_ref):
    h = jnp.maximum(jnp.dot(x_ref[...], W1[...],
                            preferred_element_type=jnp.float32) + b1[...], 0.0)
    h = jnp.maximum(jnp.dot(h, W2[...],
                            preferred_element_type=jnp.float32) + b2[...], 0.0)
    h = jnp.dot(h, W3[...], preferred_element_type=jnp.float32) + b3[...]
    y = jnp.dot(h, Wc[...], preferred_element_type=jnp.float32)
    _emit_y(y, was, wad, (y0, y1, y2, y3), as_ref, ad_ref, ms_ref, md_ref)


def _mid_body(a0, a1, a2, a3, bprev, W, was, wad,
              y0, y1, y2, y3, as_ref, ad_ref, ms_ref, md_ref):
    g = jnp.concatenate([a0[...], a1[...], a2[...], a3[...]], axis=1)
    g = jnp.maximum(g + bprev[...], 0.0)
    y = jnp.dot(g, W[...], preferred_element_type=jnp.float32)
    _emit_y(y, was, wad, (y0, y1, y2, y3), as_ref, ad_ref, ms_ref, md_ref)


def _dec_body(a0, a1, a2, a3, bprev, W1, b1, W2, b2, W3, b3, out_ref):
    g = jnp.concatenate([a0[...], a1[...], a2[...], a3[...]], axis=1)
    g = jnp.maximum(g + bprev[...], 0.0)
    h = jnp.maximum(jnp.dot(g, W1[...],
                            preferred_element_type=jnp.float32) + b1[...], 0.0)
    h = jnp.maximum(jnp.dot(h, W2[...],
                            preferred_element_type=jnp.float32) + b2[...], 0.0)
    h = jnp.dot(h, W3[...], preferred_element_type=jnp.float32) + b3[...]
    out_ref[...] = jax.nn.sigmoid(h)


def _full2d(shape):
    return pl.BlockSpec(shape, lambda i: (0, 0))


def _full1d(n):
    return pl.BlockSpec((n,), lambda i: (0,))


def _row_spec(bn, d):
    return pl.BlockSpec((bn, d), lambda i: (i, 0))


def _vec_spec(bn):
    return pl.BlockSpec((bn, 1), lambda i: (i, 0))


def _scalar_spec():
    return pl.BlockSpec((1, 1), lambda i: (0, 0))


def _y_out_specs(n):
    qs = [_row_spec(_BN, _Q)] * 4
    qt = [jax.ShapeDtypeStruct((n, _Q), jnp.float32)] * 4
    return (qs + [_vec_spec(_BN), _vec_spec(_BN),
                  _scalar_spec(), _scalar_spec()],
            qt + [jax.ShapeDtypeStruct((n, 1), jnp.float32),
                  jax.ShapeDtypeStruct((n, 1), jnp.float32),
                  jax.ShapeDtypeStruct((1, 1), jnp.float32),
                  jax.ShapeDtypeStruct((1, 1), jnp.float32)])


def _enc_call(x, W1, b1, W2, b2, W3, b3, Wc, was, wad):
    n, d = x.shape
    h = W1.shape[1]
    out_specs, out_shape = _y_out_specs(n)
    return pl.pallas_call(
        _enc_body,
        grid=(n // _BN,),
        in_specs=[_row_spec(_BN, d), _full2d((d, h)), _full1d(h),
                  _full2d((h, h)), _full1d(h), _full2d((h, h)), _full1d(h),
                  _full2d((h, h)), _full1d(h), _full1d(h)],
        out_specs=out_specs,
        out_shape=out_shape,
    )(x, W1, b1, W2, b2, W3, b3, Wc, was, wad)


def _mid_call(aq, bprev, W, was, wad):
    n = aq[0].shape[0]
    h = W.shape[0]
    out_specs, out_shape = _y_out_specs(n)
    return pl.pallas_call(
        _mid_body,
        grid=(n // _BN,),
        in_specs=[_row_spec(_BN, _Q)] * 4 + [_full1d(h), _full2d((h, h)),
                                             _full1d(h), _full1d(h)],
        out_specs=out_specs,
        out_shape=out_shape,
    )(*aq, bprev, W, was, wad)


def _dec_call(aq, bprev, W1, b1, W2, b2, W3, b3):
    n = aq[0].shape[0]
    h = W1.shape[0]
    d = W3.shape[1]
    return pl.pallas_call(
        _dec_body,
        grid=(n // _BN,),
        in_specs=[_row_spec(_BN, _Q)] * 4 + [
            _full1d(h), _full2d((h, h)), _full1d(h),
            _full2d((h, h)), _full1d(h), _full2d((h, d)), _full1d(d)],
        out_specs=[_row_spec(_BN, d)],
        out_shape=[jax.ShapeDtypeStruct((n, d), jnp.float32)],
    )(*aq, bprev, W1, b1, W2, b2, W3, b3)[0]


def _sc_edge(yq, asp, adp, mvec, srcb, dstb):
    ct = srcb.shape[1]
    rpt = _NPAD // _TILES
    mesh = plsc.VectorSubcoreMesh(core_axis_name="c", subcore_axis_name="s")

    @functools.partial(
        pl.kernel, mesh=mesh,
        out_type=jax.ShapeDtypeStruct((4, _NPAD, _Q), jnp.float32),
        compiler_params=pltpu.CompilerParams(needs_layout_passes=False,
                                             use_tc_tiling_on_sc=False),
        scratch_types=[
            pltpu.VMEM((_NPAD,), jnp.float32),
            pltpu.VMEM((_NPAD,), jnp.float32),
            pltpu.VMEM((16,), jnp.float32),
            pltpu.VMEM((ct, _CW), jnp.int32),
            pltpu.VMEM((ct, _CW), jnp.int32),
            pltpu.VMEM((ct, _CW), jnp.float32),
            pltpu.VMEM((2 * _CW, _Q), jnp.float32),
            pltpu.VMEM((_NPAD,), jnp.float32),
            pltpu.VMEM((rpt,), jnp.float32),
            pltpu.VMEM_SHARED((_NPAD, _Q), jnp.float32),
            pltpu.VMEM_SHARED((_NPAD,), jnp.float32),
            pltpu.SemaphoreType.DMA,
            pltpu.SemaphoreType.DMA,
        ])
    def k(y0_h, y1_h, y2_h, y3_h, as_h, ad_h, m_h, srcb_h, dstb_h, out_h,
          asrc_v, adst_v, m_v, src_v, dst_v, ex_v, rows_v, den_v,
          zden_v, out_sh, den_sh, gsem, ssem):
        c = lax.axis_index("c")
        s = lax.axis_index("s")
        pltpu.sync_copy(as_h, asrc_v)
        pltpu.sync_copy(ad_h, adst_v)
        pltpu.sync_copy(m_h, m_v)
        pltpu.sync_copy(srcb_h.at[s], src_v)
        pltpu.sync_copy(dstb_h.at[s], dst_v)

        zero16 = jnp.zeros((16,), jnp.float32)
        base = s * rpt
        nb = next(d for d in (2, 1) if ct % d == 0)

        def zrows(j, carry):
            for kk in range(_Q // 16):
                rows_v[j, pl.ds(kk * 16, 16)] = zero16
            return carry

        def zero_my_slice():
            lax.fori_loop(0, nb * _CW, zrows, 0)
            for t in range(rpt // _CW):
                pltpu.sync_copy(rows_v.at[pl.ds(0, _CW)],
                                out_sh.at[pl.ds(base + t * _CW, _CW)])

        zero_my_slice()
        for t in range(rpt // 16):
            zden_v[pl.ds(t * 16, 16)] = zero16
        pltpu.sync_copy(zden_v, den_sh.at[pl.ds(base, rpt)])
        plsc.subcore_barrier()

        mraw = m_v[...]
        mm = jnp.maximum(mraw, 0.2 * mraw)

        def ph1c(g, carry):
            for kk in range(8):
                si = src_v[g, pl.ds(kk * 16, 16)]
                di = dst_v[g, pl.ds(kk * 16, 16)]
                e = (plsc.load_gather(asrc_v, [si]) +
                     plsc.load_gather(adst_v, [di]))
                e = jnp.maximum(e, 0.2 * e)
                ex_v[g, pl.ds(kk * 16, 16)] = jnp.exp(e - mm)
            return carry
        lax.fori_loop(0, ct, ph1c, 0)

        kd = next(d for d in range(6, 0, -1) if ct % d == 0)
        def ph1s(t, carry):
            g0 = t * kd
            for b in range(kd):
                pltpu.async_copy(ex_v.at[g0 + b], den_sh.at[dst_v.at[g0 + b]],
                                 ssem, add=True)
            for b in range(kd):
                pltpu.make_async_copy(ex_v.at[g0 + b],
                                      den_sh.at[dst_v.at[g0 + b]],
                                      ssem).wait()
            return carry
        lax.fori_loop(0, ct // kd, ph1s, 0)
        plsc.subcore_barrier()
        pltpu.sync_copy(den_sh, den_v)

        def agg_pass(y_ref, first):
            def macro(t, carry):
                g0 = t * nb

                @pl.when(t > 0)
                def _():
                    for b in range(nb):
                        pltpu.make_async_copy(
                            rows_v.at[pl.ds(b * _CW, _CW)],
                            out_sh.at[dst_v.at[g0 + b]], ssem).wait()
                for b in range(nb):
                    pltpu.async_copy(y_ref.at[src_v.at[g0 + b]],
                                     rows_v.at[pl.ds(b * _CW, _CW)], gsem)
                if first:
                    for b in range(nb):
                        for kk in range(8):
                            di = dst_v[g0 + b, pl.ds(kk * 16, 16)]
                            dg = plsc.load_gather(den_v, [di])
                            ex_v[g0 + b, pl.ds(kk * 16, 16)] = (
                                ex_v[g0 + b, pl.ds(kk * 16, 16)]
                                / (dg + 1e-16))
                for b in range(nb):
                    pltpu.make_async_copy(
                        y_ref.at[src_v.at[g0 + b]],
                        rows_v.at[pl.ds(b * _CW, _CW)], gsem).wait()
                for b in range(nb):
                    def scale(j, cc):
                        gs = jnp.broadcast_to(g0 + b, (16,)).astype(jnp.int32)
                        js = jnp.broadcast_to(j, (16,)).astype(jnp.int32)
                        cf = plsc.load_gather(ex_v, [gs, js])
                        for kk in range(_Q // 16):
                            rows_v[b * _CW + j, pl.ds(kk * 16, 16)] = (
                                rows_v[b * _CW + j, pl.ds(kk * 16, 16)] * cf)
                        return cc
                    lax.fori_loop(0, _CW, scale, 0)
                for b in range(nb):
                    pltpu.async_copy(rows_v.at[pl.ds(b * _CW, _CW)],
                                     out_sh.at[dst_v.at[g0 + b]], ssem,
                                     add=True)
                return carry
            lax.fori_loop(0, ct // nb, macro, 0)
            for b in range(nb):
                pltpu.make_async_copy(rows_v.at[pl.ds(b * _CW, _CW)],
                                      out_sh.at[dst_v.at[b]], ssem).wait()

        def export(qi):
            pltpu.sync_copy(out_sh.at[pl.ds(base, rpt)],
                            out_h.at[qi, pl.ds(base, rpt)])

        def two_quarters(y_lo, y_hi, qi0):
            agg_pass(y_lo, True)
            plsc.subcore_barrier()
            export(qi0)
            zero_my_slice()
            plsc.subcore_barrier()
            agg_pass(y_hi, False)
            plsc.subcore_barrier()
            export(qi0 + 1)

        @pl.when(c == 0)
        def _():
            two_quarters(y0_h, y1_h, 0)

        @pl.when(c == 1)
        def _():
            two_quarters(y2_h, y3_h, 2)

    return k(*yq, asp, adp, mvec, srcb, dstb)


def kernel(x, edge_index, enc_W1, enc_b1, enc_W2, enc_b2, enc_W3, enc_b3,
           conv_W, conv_att_src, conv_att_dst, conv_b,
           dec_W1, dec_b1, dec_W2, dec_b2, dec_W3, dec_b3):
    n = x.shape[0]
    L = conv_W.shape[0]
    loop = jnp.arange(n, dtype=edge_index.dtype)
    src = jnp.concatenate([edge_index[0], loop]).astype(jnp.int32)
    dst = jnp.concatenate([edge_index[1], loop]).astype(jnp.int32)
    ne = src.shape[0]
    ct = -(-ne // (_TILES * _CW))
    epad = _TILES * ct * _CW
    srcb = jnp.pad(src, (0, epad - ne)).reshape(_TILES, ct, _CW)
    dstb = jnp.pad(dst, (0, epad - ne),
                   constant_values=n).reshape(_TILES, ct, _CW)

    *yq, a_s, a_d, ms, md = _enc_call(
        x, enc_W1, enc_b1, enc_W2, enc_b2, enc_W3, enc_b3,
        conv_W[0], conv_att_src[0], conv_att_dst[0])
    for i in range(L):
        asp = jnp.pad(a_s[:, 0], (0, _NPAD - n))
        adp = jnp.pad(a_d[:, 0], (0, _NPAD - n))
        mvec = jnp.full((16,), ms[0, 0] + md[0, 0], jnp.float32)
        out_q = _sc_edge(yq, asp, adp, mvec, srcb, dstb)
        aq = [out_q[q, :n] for q in range(4)]
        if i < L - 1:
            *yq, a_s, a_d, ms, md = _mid_call(
                aq, conv_b[i], conv_W[i + 1],
                conv_att_src[i + 1], conv_att_dst[i + 1])
    return _dec_call(aq, conv_b[L - 1], dec_W1, dec_b1, dec_W2, dec_b2,
                     dec_W3, dec_b3)
